# Initial kernel scaffold; baseline (speedup 1.0000x reference)
import jax
import jax.numpy as jnp
from jax.experimental import pallas as pl


def kernel(x, edges, W, dense_kernel, dense_bias):
    raise NotImplementedError("write your pallas kernel here")


# speedup vs baseline: 8.2440x; 8.2440x over previous
import jax
import jax.numpy as jnp
from jax import lax
from jax.experimental import pallas as pl
from jax.experimental.pallas import tpu as pltpu
from jax.experimental.pallas import tpu_sc as plsc

S = 4096
F = 256
NC = 2
NS = 16
L = 16
HALF = S // NC
FG = F // NS
CHUNK = 128
NCHUNK = HALF // CHUNK


def _sc_body(x_hbm, recv_hbm, out_p, out_cnt, recv_v, x_v, z_v, cnt_v, acc_sh):
    h = lax.axis_index("c")
    s = lax.axis_index("s")
    pltpu.sync_copy(recv_hbm.at[h], recv_v)
    pltpu.sync_copy(x_hbm.at[pl.ds(h * HALF, HALF), pl.ds(s * FG, FG)], x_v)
    def _zero(i, c):
        z_v[i, :] = jnp.zeros((L,), jnp.float32)
        return c
    lax.fori_loop(0, F, _zero, 0)
    pltpu.sync_copy(z_v, acc_sh.at[s])
    for j in range(NCHUNK):
        pltpu.sync_copy(
            x_v.at[pl.ds(j * CHUNK, CHUNK), :],
            acc_sh.at[s].at[recv_v.at[j]],
            add=True,
        )
    for i in range(F // L):
        cnt_v[pl.ds(i * L, L)] = jnp.zeros((L,), jnp.float32)
    ones = jnp.ones((L,), jnp.float32)
    for i in range(CHUNK // L):
        idx = recv_v[s, pl.ds(i * L, L)]
        plsc.addupdate_scatter(cnt_v, [idx], ones)
    pltpu.sync_copy(acc_sh.at[s], out_p.at[h, :, pl.ds(s * FG, FG)])
    pltpu.sync_copy(cnt_v, out_cnt.at[h * NS + s])


_sc_scatter = pl.kernel(
    _sc_body,
    out_type=[
        jax.ShapeDtypeStruct((NC, F, F), jnp.float32),
        jax.ShapeDtypeStruct((NC * NS, F), jnp.float32),
    ],
    mesh=plsc.VectorSubcoreMesh(core_axis_name="c", subcore_axis_name="s"),
    compiler_params=pltpu.CompilerParams(
        use_tc_tiling_on_sc=False, needs_layout_passes=False
    ),
    scratch_types=[
        pltpu.VMEM((NS, CHUNK), jnp.int32),
        pltpu.VMEM((HALF, FG), jnp.float32),
        pltpu.VMEM((F, FG), jnp.float32),
        pltpu.VMEM((F,), jnp.float32),
        pltpu.VMEM_SHARED((NS, F, FG), jnp.float32),
    ],
)


def _tc_body(p_ref, cnt_ref, w_ref, dk_ref, o_ref):
    xs = p_ref[0] + p_ref[1]
    pm = jnp.dot(xs, w_ref[...], preferred_element_type=jnp.float32)
    r = jnp.maximum(pm, 0.0)
    c = jnp.sum(cnt_ref[...], axis=0, keepdims=True)
    tot = jnp.sum(r * c * dk_ref[...])
    o_ref[...] = jnp.reshape(tot, (1, 1))


_tc_finish = pl.pallas_call(
    _tc_body,
    out_shape=jax.ShapeDtypeStruct((1, 1), jnp.float32),
)


def kernel(x, edges, W, dense_kernel, dense_bias):
    x = x.astype(jnp.float32)
    recv = jnp.concatenate([edges[:, 1], edges[:, 0]]).reshape(NC, NS, CHUNK)
    parts, cnts = _sc_scatter(x, recv)
    out = _tc_finish(parts, cnts, W, dense_kernel[:F])
    return out[0, 0] + dense_bias[0]


# speedup vs baseline: 9.1414x; 1.1089x over previous
import jax
import jax.numpy as jnp
from jax import lax
from jax.experimental import pallas as pl
from jax.experimental.pallas import tpu as pltpu
from jax.experimental.pallas import tpu_sc as plsc

S = 4096
F = 256
NC = 2
NS = 16
L = 16
RG = 8
FGN = 4
RROWS = S // RG
FG = F // FGN
CHUNK = 128
NCHUNK = RROWS // CHUNK


def _sc_body(x_hbm, recv_hbm, out_p, out_cnt,
             recv_v, x_v, z_v, cnt_v, acc_sh, sem_in):
    wid = lax.axis_index("s") * NC + lax.axis_index("c")
    rg = wid // FGN
    fg = wid % FGN
    sid = lax.axis_index("s")
    cp_r = pltpu.async_copy(recv_hbm.at[pl.ds(rg * NCHUNK, NCHUNK)], recv_v,
                            sem_in)
    cp_x = pltpu.async_copy(
        x_hbm.at[pl.ds(rg * RROWS, RROWS), pl.ds(fg * FG, FG)], x_v, sem_in)

    def _zero(i, c):
        for k in range(FG // L):
            z_v[i, pl.ds(k * L, L)] = jnp.zeros((L,), jnp.float32)
        return c
    lax.fori_loop(0, F, _zero, 0)
    for i in range(F // L):
        cnt_v[pl.ds(i * L, L)] = jnp.zeros((L,), jnp.float32)
    pltpu.sync_copy(z_v, acc_sh.at[sid])
    cp_r.wait()
    cp_x.wait()
    for j in range(NCHUNK):
        pltpu.sync_copy(
            x_v.at[pl.ds(j * CHUNK, CHUNK), :],
            acc_sh.at[sid].at[recv_v.at[j]],
            add=True,
        )
    ones = jnp.ones((L,), jnp.float32)
    for i in range(CHUNK // L):
        idx = recv_v[fg, pl.ds(i * L, L)]
        plsc.addupdate_scatter(cnt_v, [idx], ones)
    pltpu.sync_copy(acc_sh.at[sid], out_p.at[rg, :, pl.ds(fg * FG, FG)])
    pltpu.sync_copy(cnt_v, out_cnt.at[wid])


_sc_scatter = pl.kernel(
    _sc_body,
    out_type=[
        jax.ShapeDtypeStruct((RG, F, F), jnp.float32),
        jax.ShapeDtypeStruct((NC * NS, F), jnp.float32),
    ],
    mesh=plsc.VectorSubcoreMesh(core_axis_name="c", subcore_axis_name="s"),
    compiler_params=pltpu.CompilerParams(
        use_tc_tiling_on_sc=False, needs_layout_passes=False
    ),
    scratch_types=[
        pltpu.VMEM((NCHUNK, CHUNK), jnp.int32),
        pltpu.VMEM((RROWS, FG), jnp.float32),
        pltpu.VMEM((F, FG), jnp.float32),
        pltpu.VMEM((F,), jnp.float32),
        pltpu.VMEM_SHARED((NS, F, FG), jnp.float32),
        pltpu.SemaphoreType.DMA,
    ],
)


def _tc_body(p_ref, cnt_ref, w_ref, dk_ref, o_ref):
    xs = jnp.sum(p_ref[...], axis=0)
    pm = jnp.dot(xs, w_ref[...], preferred_element_type=jnp.float32)
    r = jnp.maximum(pm, 0.0)
    c = jnp.sum(cnt_ref[...], axis=0, keepdims=True)
    tot = jnp.sum(r * c * dk_ref[...])
    o_ref[...] = jnp.reshape(tot, (1, 1))


_tc_finish = pl.pallas_call(
    _tc_body,
    out_shape=jax.ShapeDtypeStruct((1, 1), jnp.float32),
)


def kernel(x, edges, W, dense_kernel, dense_bias):
    x = x.astype(jnp.float32)
    recv = jnp.concatenate([edges[:, 1], edges[:, 0]]).reshape(S // CHUNK,
                                                               CHUNK)
    parts, cnts = _sc_scatter(x, recv)
    out = _tc_finish(parts, cnts, W, dense_kernel[:F])
    return out[0, 0] + dense_bias[0]


# speedup vs baseline: 9.4712x; 1.0361x over previous
import jax
import jax.numpy as jnp
from jax import lax
from jax.experimental import pallas as pl
from jax.experimental.pallas import tpu as pltpu
from jax.experimental.pallas import tpu_sc as plsc

S = 4096
E = S // 2
F = 256
NC = 2
NS = 16
L = 16
CHUNK = S // (NC * NS)
STRIPE = F // NS


def _sc_body(x_hbm, edges_hbm, out_p, out_cnt,
             e_v, recv_v, x_v, z_v, cnt_v, acc_sh, sem_in):
    c = lax.axis_index("c")
    sid = lax.axis_index("s")
    w = c * NS + sid
    er0 = lax.rem(w, NS) * CHUNK
    cp_e = pltpu.async_copy(edges_hbm.at[pl.ds(er0, CHUNK), :], e_v, sem_in)
    cp_x = pltpu.async_copy(x_hbm.at[pl.ds(w * CHUNK, CHUNK), :], x_v, sem_in)
    zero = jnp.zeros((L,), jnp.float32)
    for i in range(STRIPE):
        for k in range(F // L):
            z_v[i, pl.ds(k * L, L)] = zero
    for i in range(F // L):
        cnt_v[pl.ds(i * L, L)] = zero
    cp_e.wait()
    col = jnp.where(w < NS, jnp.int32(1), jnp.int32(0))
    cols = jnp.zeros((L,), jnp.int32) + col
    for i in range(CHUNK // L):
        rows = lax.iota(jnp.int32, L) + jnp.int32(i * L)
        recv_v[pl.ds(i * L, L)] = plsc.load_gather(e_v, [rows, cols])
    pltpu.sync_copy(z_v, acc_sh.at[pl.ds(sid * STRIPE, STRIPE), :])
    plsc.subcore_barrier()
    cp_x.wait()
    pltpu.sync_copy(x_v, acc_sh.at[recv_v], add=True)
    ones = jnp.ones((L,), jnp.float32)
    for i in range(CHUNK // L):
        idx = recv_v[pl.ds(i * L, L)]
        plsc.addupdate_scatter(cnt_v, [idx], ones)
    pltpu.sync_copy(cnt_v, out_cnt.at[w])
    plsc.subcore_barrier()
    pltpu.sync_copy(acc_sh.at[pl.ds(sid * STRIPE, STRIPE), :],
                    out_p.at[c, pl.ds(sid * STRIPE, STRIPE), :])


_sc_scatter = pl.kernel(
    _sc_body,
    out_type=[
        jax.ShapeDtypeStruct((NC, F, F), jnp.float32),
        jax.ShapeDtypeStruct((NC * NS, F), jnp.float32),
    ],
    mesh=plsc.VectorSubcoreMesh(core_axis_name="c", subcore_axis_name="s"),
    compiler_params=pltpu.CompilerParams(
        use_tc_tiling_on_sc=False, needs_layout_passes=False
    ),
    scratch_types=[
        pltpu.VMEM((CHUNK, 2), jnp.int32),
        pltpu.VMEM((CHUNK,), jnp.int32),
        pltpu.VMEM((CHUNK, F), jnp.float32),
        pltpu.VMEM((STRIPE, F), jnp.float32),
        pltpu.VMEM((F,), jnp.float32),
        pltpu.VMEM_SHARED((F, F), jnp.float32),
        pltpu.SemaphoreType.DMA,
    ],
)


def _tc_body(p_ref, cnt_ref, w_ref, dk_ref, b_ref, o_ref):
    xs = p_ref[0] + p_ref[1]
    pm = jnp.dot(xs, w_ref[...], preferred_element_type=jnp.float32)
    r = jnp.maximum(pm, 0.0)
    c = jnp.sum(cnt_ref[...], axis=0, keepdims=True)
    tot = jnp.sum(r * c * dk_ref[0:F, :])
    o_ref[...] = jnp.reshape(tot + b_ref[0], (1, 1))


_tc_finish = pl.pallas_call(
    _tc_body,
    out_shape=jax.ShapeDtypeStruct((1, 1), jnp.float32),
    in_specs=[
        pl.BlockSpec(memory_space=pltpu.MemorySpace.VMEM),
        pl.BlockSpec(memory_space=pltpu.MemorySpace.VMEM),
        pl.BlockSpec(memory_space=pltpu.MemorySpace.VMEM),
        pl.BlockSpec(memory_space=pltpu.MemorySpace.VMEM),
        pl.BlockSpec(memory_space=pltpu.MemorySpace.SMEM),
    ],
)


def kernel(x, edges, W, dense_kernel, dense_bias):
    parts, cnts = _sc_scatter(x.astype(jnp.float32), edges)
    out = _tc_finish(parts, cnts, W, dense_kernel, dense_bias)
    return jnp.reshape(out, ())


# speedup vs baseline: 10.2526x; 1.0825x over previous
import jax
import jax.numpy as jnp
from jax import lax
from jax.experimental import pallas as pl
from jax.experimental.pallas import tpu as pltpu
from jax.experimental.pallas import tpu_sc as plsc

S = 4096
E = S // 2
F = 256
NC = 2
NS = 16
NW = NC * NS
L = 16
CHUNK = S // NW
STRIPE = F // NS


def _sc_body(x_hbm, edges_hbm, out_p, out_cnt,
             e_v, recv_v, x_v, z_v, cnt_v, acc_sh, sem_in):
    c = lax.axis_index("c")
    sid = lax.axis_index("s")
    w = c * NS + sid
    e0 = lax.rem(w, NS) * (2 * CHUNK)
    cp_e = pltpu.async_copy(edges_hbm.at[pl.ds(e0, 2 * CHUNK)], e_v, sem_in)
    cp_x = pltpu.async_copy(x_hbm.at[pl.ds(w * CHUNK, CHUNK), :], x_v, sem_in)
    zero = jnp.zeros((L,), jnp.float32)
    for i in range(STRIPE):
        for k in range(F // L):
            z_v[i, pl.ds(k * L, L)] = zero
    for i in range(STRIPE):
        for k in range(128 // L):
            cnt_v[i, pl.ds(k * L, L)] = zero
    cp_e.wait()
    col = jnp.where(w < NS, jnp.int32(1), jnp.int32(0))
    cols = jnp.zeros((L,), jnp.int32) + col
    for i in range(CHUNK // L):
        rows = lax.iota(jnp.int32, L) + jnp.int32(i * L)
        recv_v[pl.ds(i * L, L)] = plsc.load_gather(e_v, [rows * 2 + cols])
    pltpu.sync_copy(z_v, acc_sh.at[pl.ds(sid * STRIPE, STRIPE), :])
    plsc.subcore_barrier()
    cp_x.wait()
    pltpu.sync_copy(x_v, acc_sh.at[recv_v], add=True)
    ones = jnp.ones((L,), jnp.float32)
    for i in range(CHUNK // L):
        f = recv_v[pl.ds(i * L, L)]
        plsc.addupdate_scatter(
            cnt_v, [(f // 128) * 8, lax.rem(f, 128)], ones)
    pltpu.sync_copy(cnt_v, out_cnt.at[pl.ds(w * STRIPE, STRIPE), :])
    plsc.subcore_barrier()
    pltpu.sync_copy(acc_sh.at[pl.ds(sid * STRIPE, STRIPE), pl.ds(0, 128)],
                    out_p.at[c, 0, pl.ds(sid * STRIPE, STRIPE), :])
    pltpu.sync_copy(acc_sh.at[pl.ds(sid * STRIPE, STRIPE), pl.ds(128, 128)],
                    out_p.at[c, 1, pl.ds(sid * STRIPE, STRIPE), :])


_sc_scatter = pl.kernel(
    _sc_body,
    out_type=[
        jax.ShapeDtypeStruct((NC, 2, F, 128), jnp.float32),
        jax.ShapeDtypeStruct((NW * STRIPE, 128), jnp.float32),
    ],
    mesh=plsc.VectorSubcoreMesh(core_axis_name="c", subcore_axis_name="s"),
    compiler_params=pltpu.CompilerParams(
        use_tc_tiling_on_sc=False, needs_layout_passes=False
    ),
    scratch_types=[
        pltpu.VMEM((2 * CHUNK,), jnp.int32),
        pltpu.VMEM((CHUNK,), jnp.int32),
        pltpu.VMEM((CHUNK, F), jnp.float32),
        pltpu.VMEM((STRIPE, F), jnp.float32),
        pltpu.VMEM((STRIPE, 128), jnp.float32),
        pltpu.VMEM_SHARED((F, F), jnp.float32),
        pltpu.SemaphoreType.DMA,
    ],
)


def _tc_body(p_ref, cnt_ref, w_ref, dk_ref, b_ref, o_ref):
    xs = jnp.concatenate(
        [p_ref[0, 0] + p_ref[1, 0], p_ref[0, 1] + p_ref[1, 1]], axis=1)
    pm = jnp.dot(xs, w_ref[...], preferred_element_type=jnp.float32)
    r = jnp.maximum(pm, 0.0)
    s = jnp.sum(jnp.reshape(cnt_ref[...], (NW, STRIPE, 128)), axis=0)
    cnt = jnp.concatenate([s[0:1, :], s[8:9, :]], axis=1)
    tot = jnp.sum(r * cnt * dk_ref[0:F, :])
    o_ref[...] = jnp.reshape(tot + b_ref[0], (1, 1))


_tc_finish = pl.pallas_call(
    _tc_body,
    out_shape=jax.ShapeDtypeStruct((1, 1), jnp.float32),
    in_specs=[
        pl.BlockSpec(memory_space=pltpu.MemorySpace.VMEM),
        pl.BlockSpec(memory_space=pltpu.MemorySpace.VMEM),
        pl.BlockSpec(memory_space=pltpu.MemorySpace.VMEM),
        pl.BlockSpec(memory_space=pltpu.MemorySpace.VMEM),
        pl.BlockSpec(memory_space=pltpu.MemorySpace.SMEM),
    ],
)


def kernel(x, edges, W, dense_kernel, dense_bias):
    parts, cnts = _sc_scatter(x.astype(jnp.float32), jnp.ravel(edges))
    out = _tc_finish(parts, cnts, W, dense_kernel, dense_bias)
    return jnp.reshape(out, ())


# speedup vs baseline: 10.4020x; 1.0146x over previous
import jax
import jax.numpy as jnp
from jax import lax
from jax.experimental import pallas as pl
from jax.experimental.pallas import tpu as pltpu
from jax.experimental.pallas import tpu_sc as plsc

S = 4096
E = S // 2
F = 256
NC = 2
NS = 16
NW = NC * NS
L = 16
CHUNK = S // NW
STRIPE = F // NS


def _sc_body(x_hbm, edges_hbm, out_p, out_cnt,
             e_v, recv_v, x_v, z_v, cnt_v, acc_sh, sem_in, sem_x0, sem_x1):
    c = lax.axis_index("c")
    sid = lax.axis_index("s")
    w = c * NS + sid
    e0 = lax.rem(w, NS) * (2 * CHUNK)
    cp_e = pltpu.async_copy(edges_hbm.at[pl.ds(e0, 2 * CHUNK)], e_v, sem_in)
    half = CHUNK // 2
    cp_x0 = pltpu.async_copy(
        x_hbm.at[pl.ds(w * CHUNK, half), :], x_v.at[pl.ds(0, half), :],
        sem_x0)
    cp_x1 = pltpu.async_copy(
        x_hbm.at[pl.ds(w * CHUNK + half, half), :],
        x_v.at[pl.ds(half, half), :], sem_x1)
    zero = jnp.zeros((L,), jnp.float32)

    def _zero(i, carry):
        for k in range(F // L):
            z_v[i, pl.ds(k * L, L)] = zero
        return carry
    lax.fori_loop(0, STRIPE, _zero, 0)
    for r in (0, 8):
        for k in range(128 // L):
            cnt_v[r, pl.ds(k * L, L)] = zero
    cp_e.wait()
    col = jnp.where(w < NS, jnp.int32(1), jnp.int32(0))
    cols = jnp.zeros((L,), jnp.int32) + col
    for i in range(CHUNK // L):
        rows = lax.iota(jnp.int32, L) + jnp.int32(i * L)
        recv_v[i // 4, pl.ds((i % 4) * L, L)] = plsc.load_gather(
            e_v, [rows * 2 + cols])
    pltpu.sync_copy(z_v, acc_sh.at[pl.ds(sid * STRIPE, STRIPE), :])
    plsc.subcore_barrier()
    cp_x0.wait()
    pltpu.sync_copy(x_v.at[pl.ds(0, half), :], acc_sh.at[recv_v.at[0]],
                    add=True)
    cp_x1.wait()
    pltpu.sync_copy(x_v.at[pl.ds(half, half), :], acc_sh.at[recv_v.at[1]],
                    add=True)
    ones = jnp.ones((L,), jnp.float32)
    for i in range(CHUNK // L):
        f = recv_v[i // 4, pl.ds((i % 4) * L, L)]
        plsc.addupdate_scatter(
            cnt_v, [(f // 128) * 8, lax.rem(f, 128)], ones)
    pltpu.sync_copy(cnt_v, out_cnt.at[pl.ds(w * STRIPE, STRIPE), :])
    plsc.subcore_barrier()
    pltpu.sync_copy(acc_sh.at[pl.ds(sid * STRIPE, STRIPE), pl.ds(0, 128)],
                    out_p.at[c, 0, pl.ds(sid * STRIPE, STRIPE), :])
    pltpu.sync_copy(acc_sh.at[pl.ds(sid * STRIPE, STRIPE), pl.ds(128, 128)],
                    out_p.at[c, 1, pl.ds(sid * STRIPE, STRIPE), :])


_sc_scatter = pl.kernel(
    _sc_body,
    out_type=[
        jax.ShapeDtypeStruct((NC, 2, F, 128), jnp.float32),
        jax.ShapeDtypeStruct((NW * STRIPE, 128), jnp.float32),
    ],
    mesh=plsc.VectorSubcoreMesh(core_axis_name="c", subcore_axis_name="s"),
    compiler_params=pltpu.CompilerParams(
        use_tc_tiling_on_sc=False, needs_layout_passes=False
    ),
    scratch_types=[
        pltpu.VMEM((2 * CHUNK,), jnp.int32),
        pltpu.VMEM((2, CHUNK // 2), jnp.int32),
        pltpu.VMEM((CHUNK, F), jnp.float32),
        pltpu.VMEM((STRIPE, F), jnp.float32),
        pltpu.VMEM((STRIPE, 128), jnp.float32),
        pltpu.VMEM_SHARED((F, F), jnp.float32),
        pltpu.SemaphoreType.DMA,
        pltpu.SemaphoreType.DMA,
        pltpu.SemaphoreType.DMA,
    ],
)


def _tc_body(p_ref, cnt_ref, w_ref, dk_ref, b_ref, o_ref):
    xs = jnp.concatenate(
        [p_ref[0, 0] + p_ref[1, 0], p_ref[0, 1] + p_ref[1, 1]], axis=1)
    pm = jnp.dot(xs, w_ref[...], preferred_element_type=jnp.float32)
    r = jnp.maximum(pm, 0.0)
    s = jnp.sum(jnp.reshape(cnt_ref[...], (NW, STRIPE, 128)), axis=0)
    cnt = jnp.concatenate([s[0:1, :], s[8:9, :]], axis=1)
    tot = jnp.sum(r * cnt * dk_ref[0:F, :])
    o_ref[...] = jnp.reshape(tot + b_ref[0], (1, 1))


_tc_finish = pl.pallas_call(
    _tc_body,
    out_shape=jax.ShapeDtypeStruct((1, 1), jnp.float32),
    in_specs=[
        pl.BlockSpec(memory_space=pltpu.MemorySpace.VMEM),
        pl.BlockSpec(memory_space=pltpu.MemorySpace.VMEM),
        pl.BlockSpec(memory_space=pltpu.MemorySpace.VMEM),
        pl.BlockSpec(memory_space=pltpu.MemorySpace.VMEM),
        pl.BlockSpec(memory_space=pltpu.MemorySpace.SMEM),
    ],
)


def kernel(x, edges, W, dense_kernel, dense_bias):
    parts, cnts = _sc_scatter(x.astype(jnp.float32), jnp.ravel(edges))
    out = _tc_finish(parts, cnts, W, dense_kernel, dense_bias)
    return jnp.reshape(out, ())
